# baseline (speedup 1.0000x reference)
import functools

import jax
import jax.numpy as jnp
from jax import lax
from jax.experimental import pallas as pl
from jax.experimental.pallas import tpu as pltpu
from jax.experimental.pallas import tpu_sc as plsc

_NFIELD = 26
_NEMB = 16
_BATCH = 16384

_NW = 32
_BW = _BATCH // _NW
_NGRP = _BW // _NEMB

_NFEAT = 1000000
_DBW = 512
_NBLK_FULL = _NFEAT // _DBW
_BPW = _NBLK_FULL // _NW
_NEXTRA = _NBLK_FULL - _BPW * _NW
_TAIL_C0 = _NBLK_FULL * _DBW


def _detile_body(tabT_hbm, tail_hbm, out_hbm, buf_v, row0_v, row1_v, sem_i, sem_o):
    c = lax.axis_index("c")
    s = lax.axis_index("s")
    wid = s * 2 + c
    blk0 = wid * _BPW
    iota16 = lax.broadcasted_iota(jnp.int32, (16,), 0)
    rows = (row0_v, row1_v)

    def fire_in(i, b):
        c0 = (blk0 + i) * _DBW
        return pltpu.async_copy(
            tabT_hbm.at[:, pl.ds(c0, _DBW)], buf_v.at[b], sem_i.at[b]
        )

    def transpose_block(src, dst, width):
        @plsc.parallel_loop(0, width, step=1, unroll=8)
        def _(j):
            col = plsc.load_gather(src, [iota16, jnp.full((16,), j, jnp.int32)])
            dst[pl.ds(j * _NEMB, _NEMB)] = col

    fire_in(0, 0)
    fire_in(1, 1)

    def body2(t, _):
        for b in range(2):
            i = t * 2 + b
            c0 = (blk0 + i) * _DBW
            pltpu.make_async_copy(
                tabT_hbm.at[:, pl.ds(0, _DBW)], buf_v.at[b], sem_i.at[b]
            ).wait()

            @pl.when(i >= 2)
            def _():
                pltpu.make_async_copy(
                    rows[b], out_hbm.at[pl.ds(0, _DBW * _NEMB)], sem_o.at[b]
                ).wait()

            transpose_block(buf_v.at[b], rows[b], _DBW)

            pltpu.async_copy(
                rows[b], out_hbm.at[pl.ds(c0 * _NEMB, _DBW * _NEMB)], sem_o.at[b]
            )

            @pl.when(i + 2 < _BPW)
            def _():
                fire_in(i + 2, b)

        return 0

    lax.fori_loop(0, _BPW // 2, body2, 0, unroll=False)
    i_last = _BPW - 1
    b_last = i_last % 2
    pltpu.make_async_copy(
        tabT_hbm.at[:, pl.ds(0, _DBW)], buf_v.at[b_last], sem_i.at[b_last]
    ).wait()
    pltpu.make_async_copy(
        rows[b_last], out_hbm.at[pl.ds(0, _DBW * _NEMB)], sem_o.at[b_last]
    ).wait()
    transpose_block(buf_v.at[b_last], rows[b_last], _DBW)
    c0_last = (blk0 + i_last) * _DBW
    pltpu.async_copy(
        rows[b_last], out_hbm.at[pl.ds(c0_last * _NEMB, _DBW * _NEMB)],
        sem_o.at[b_last],
    )
    pltpu.make_async_copy(
        rows[1 - b_last], out_hbm.at[pl.ds(0, _DBW * _NEMB)], sem_o.at[1 - b_last]
    ).wait()
    pltpu.make_async_copy(
        rows[b_last], out_hbm.at[pl.ds(0, _DBW * _NEMB)], sem_o.at[b_last]
    ).wait()

    @pl.when(wid < _NEXTRA)
    def _():
        c0 = (_NBLK_FULL - _NEXTRA + wid) * _DBW
        pltpu.sync_copy(tabT_hbm.at[:, pl.ds(c0, _DBW)], buf_v.at[0])
        transpose_block(buf_v.at[0], row0_v, _DBW)
        pltpu.sync_copy(row0_v, out_hbm.at[pl.ds(c0 * _NEMB, _DBW * _NEMB)])

    @pl.when(wid == _NEXTRA)
    def _():
        tail_sz = (_NFEAT - _TAIL_C0) * _NEMB
        pltpu.sync_copy(tail_hbm, row0_v.at[pl.ds(0, tail_sz)])
        pltpu.sync_copy(
            row0_v.at[pl.ds(0, tail_sz)],
            out_hbm.at[pl.ds(_TAIL_C0 * _NEMB, tail_sz)],
        )


def _gather_body(idxT_hbm, valT_hbm, table_hbm, out_hbm,
                 idx_v, val_v, rows_v, col_v, sem_g, sem_o):
    c = lax.axis_index("c")
    s = lax.axis_index("s")
    wid = s * 2 + c
    b0 = wid * _BW

    pltpu.sync_copy(idxT_hbm.at[:, pl.ds(b0, _BW)], idx_v)
    pltpu.sync_copy(valT_hbm.at[:, pl.ds(b0, _BW)], val_v)

    iota16 = lax.broadcasted_iota(jnp.int32, (16,), 0)

    def fire_gather(f):
        b = f % 2
        return [
            pltpu.async_copy(
                table_hbm.at[idx_v.at[f, pl.ds(k * 128, 128)]],
                rows_v.at[b, pl.ds(k * 128, 128)],
                sem_g.at[b],
            )
            for k in range(_BW // 128)
        ]

    gd = {0: fire_gather(0)}
    od = {}
    for f in range(_NFIELD):
        b = f % 2
        if f + 1 < _NFIELD:
            gd[f + 1] = fire_gather(f + 1)
        for d in gd[f]:
            d.wait()
        if f >= 2:
            od[f - 2].wait()

        rows_b = rows_v.at[b]
        col_b = col_v.at[b]

        @plsc.parallel_loop(0, _NGRP, step=1, unroll=1)
        def _(j, f=f, rows_b=rows_b, col_b=col_b):
            r0 = j * _NEMB
            vals = val_v[f, pl.ds(r0, _NEMB)]
            ridx = iota16 + r0
            for e in range(_NEMB):
                col = plsc.load_gather(rows_b, [ridx, jnp.full((16,), e, jnp.int32)])
                col_b[e, pl.ds(r0, _NEMB)] = col * vals
        od[f] = pltpu.async_copy(
            col_b, out_hbm.at[f, :, pl.ds(b0, _BW)], sem_o.at[b]
        )
    od[_NFIELD - 2].wait()
    od[_NFIELD - 1].wait()


@jax.jit
def kernel(idx, value, table):
    idxT = idx.T
    valT = value.T
    mesh = plsc.VectorSubcoreMesh(core_axis_name="c", subcore_axis_name="s")
    tab_lin = pl.kernel(
        _detile_body,
        out_type=jax.ShapeDtypeStruct((_NFEAT * _NEMB,), jnp.float32),
        mesh=mesh,
        scratch_types=[
            pltpu.VMEM((2, _NEMB, _DBW), jnp.float32),
            pltpu.VMEM((_DBW * _NEMB,), jnp.float32),
            pltpu.VMEM((_DBW * _NEMB,), jnp.float32),
            pltpu.SemaphoreType.DMA((2,)),
            pltpu.SemaphoreType.DMA((2,)),
        ],
        compiler_params=pltpu.CompilerParams(
            use_tc_tiling_on_sc=True, needs_layout_passes=False
        ),
    )(table.T, lax.slice(table, (_TAIL_C0, 0), (_NFEAT, _NEMB)).reshape(-1))
    out = pl.kernel(
        _gather_body,
        out_type=jax.ShapeDtypeStruct((_NFIELD, _NEMB, _BATCH), jnp.float32),
        mesh=mesh,
        scratch_types=[
            pltpu.VMEM((_NFIELD, _BW), jnp.int32),
            pltpu.VMEM((_NFIELD, _BW), jnp.float32),
            pltpu.VMEM((2, _BW, _NEMB), jnp.float32),
            pltpu.VMEM((2, _NEMB, _BW), jnp.float32),
            pltpu.SemaphoreType.DMA((2,)),
            pltpu.SemaphoreType.DMA((2,)),
        ],
        compiler_params=pltpu.CompilerParams(
            use_tc_tiling_on_sc=False, needs_layout_passes=False
        ),
    )(idxT, valT, tab_lin.reshape(_NFEAT, _NEMB))
    return out.transpose(2, 0, 1)


# speedup vs baseline: 1.0528x; 1.0528x over previous
import functools

import jax
import jax.numpy as jnp
from jax import lax
from jax.experimental import pallas as pl
from jax.experimental.pallas import tpu as pltpu
from jax.experimental.pallas import tpu_sc as plsc

_NFIELD = 26
_NEMB = 16
_BATCH = 16384

_NW = 32
_BW = _BATCH // _NW
_NGRP = _BW // _NEMB

_NFEAT = 1000000
_DBW = 128
_NBLK_FULL = _NFEAT // _DBW
_BPW = _NBLK_FULL // _NW
_NEXTRA = _NBLK_FULL - _BPW * _NW
_TAIL_C0 = _NBLK_FULL * _DBW


def _detile_body(tabT_hbm, tail_hbm, out_hbm, buf_v, row0_v, row1_v, sem_i, sem_o):
    c = lax.axis_index("c")
    s = lax.axis_index("s")
    wid = s * 2 + c
    blk0 = wid * _BPW
    iota16 = lax.broadcasted_iota(jnp.int32, (16,), 0)
    rows = (row0_v, row1_v)

    def fire_in(i, b):
        c0 = (blk0 + i) * _DBW
        return pltpu.async_copy(
            tabT_hbm.at[:, pl.ds(c0, _DBW)], buf_v.at[b], sem_i.at[b]
        )

    def transpose_block(src, dst, width):
        @plsc.parallel_loop(0, width, step=1, unroll=8)
        def _(j):
            col = plsc.load_gather(src, [iota16, jnp.full((16,), j, jnp.int32)])
            dst[pl.ds(j * _NEMB, _NEMB)] = col

    fire_in(0, 0)
    fire_in(1, 1)

    def body2(t, _):
        for b in range(2):
            i = t * 2 + b
            c0 = (blk0 + i) * _DBW
            pltpu.make_async_copy(
                tabT_hbm.at[:, pl.ds(0, _DBW)], buf_v.at[b], sem_i.at[b]
            ).wait()

            @pl.when(i >= 2)
            def _():
                pltpu.make_async_copy(
                    rows[b], out_hbm.at[pl.ds(0, _DBW * _NEMB)], sem_o.at[b]
                ).wait()

            transpose_block(buf_v.at[b], rows[b], _DBW)

            pltpu.async_copy(
                rows[b], out_hbm.at[pl.ds(c0 * _NEMB, _DBW * _NEMB)], sem_o.at[b]
            )

            @pl.when(i + 2 < _BPW)
            def _():
                fire_in(i + 2, b)

        return 0

    lax.fori_loop(0, _BPW // 2, body2, 0, unroll=False)
    if _BPW % 2:
        i_last = _BPW - 1
        b_last = i_last % 2
        pltpu.make_async_copy(
            tabT_hbm.at[:, pl.ds(0, _DBW)], buf_v.at[b_last], sem_i.at[b_last]
        ).wait()
        pltpu.make_async_copy(
            rows[b_last], out_hbm.at[pl.ds(0, _DBW * _NEMB)], sem_o.at[b_last]
        ).wait()
        transpose_block(buf_v.at[b_last], rows[b_last], _DBW)
        c0_last = (blk0 + i_last) * _DBW
        pltpu.async_copy(
            rows[b_last], out_hbm.at[pl.ds(c0_last * _NEMB, _DBW * _NEMB)],
            sem_o.at[b_last],
        )
        pltpu.make_async_copy(
            rows[1 - b_last], out_hbm.at[pl.ds(0, _DBW * _NEMB)],
            sem_o.at[1 - b_last],
        ).wait()
        pltpu.make_async_copy(
            rows[b_last], out_hbm.at[pl.ds(0, _DBW * _NEMB)], sem_o.at[b_last]
        ).wait()
    else:
        for b in range(2):
            pltpu.make_async_copy(
                rows[b], out_hbm.at[pl.ds(0, _DBW * _NEMB)], sem_o.at[b]
            ).wait()

    @pl.when(wid < _NEXTRA)
    def _():
        c0 = (_NBLK_FULL - _NEXTRA + wid) * _DBW
        pltpu.sync_copy(tabT_hbm.at[:, pl.ds(c0, _DBW)], buf_v.at[0])
        transpose_block(buf_v.at[0], row0_v, _DBW)
        pltpu.sync_copy(row0_v, out_hbm.at[pl.ds(c0 * _NEMB, _DBW * _NEMB)])

    @pl.when(wid == _NEXTRA)
    def _():
        tail_sz = (_NFEAT - _TAIL_C0) * _NEMB
        pltpu.sync_copy(tail_hbm, row0_v.at[pl.ds(0, tail_sz)])
        pltpu.sync_copy(
            row0_v.at[pl.ds(0, tail_sz)],
            out_hbm.at[pl.ds(_TAIL_C0 * _NEMB, tail_sz)],
        )


def _gather_body(idxT_hbm, valT_hbm, table_hbm, out_hbm,
                 idx_v, val_v, rows_v, col_v, sem_g, sem_o):
    c = lax.axis_index("c")
    s = lax.axis_index("s")
    wid = s * 2 + c
    b0 = wid * _BW

    pltpu.sync_copy(idxT_hbm.at[:, pl.ds(b0, _BW)], idx_v)
    pltpu.sync_copy(valT_hbm.at[:, pl.ds(b0, _BW)], val_v)

    iota16 = lax.broadcasted_iota(jnp.int32, (16,), 0)

    def fire_gather(f):
        b = f % 2
        return [
            pltpu.async_copy(
                table_hbm.at[idx_v.at[f, pl.ds(k * 128, 128)]],
                rows_v.at[b, pl.ds(k * 128, 128)],
                sem_g.at[b],
            )
            for k in range(_BW // 128)
        ]

    gd = {0: fire_gather(0)}
    od = {}
    for f in range(_NFIELD):
        b = f % 2
        if f + 1 < _NFIELD:
            gd[f + 1] = fire_gather(f + 1)
        for d in gd[f]:
            d.wait()
        if f >= 2:
            od[f - 2].wait()

        rows_b = rows_v.at[b]
        col_b = col_v.at[b]

        @plsc.parallel_loop(0, _NGRP, step=1, unroll=1)
        def _(j, f=f, rows_b=rows_b, col_b=col_b):
            r0 = j * _NEMB
            vals = val_v[f, pl.ds(r0, _NEMB)]
            ridx = iota16 + r0
            for e in range(_NEMB):
                col = plsc.load_gather(rows_b, [ridx, jnp.full((16,), e, jnp.int32)])
                col_b[e, pl.ds(r0, _NEMB)] = col * vals
        od[f] = pltpu.async_copy(
            col_b, out_hbm.at[f, :, pl.ds(b0, _BW)], sem_o.at[b]
        )
    od[_NFIELD - 2].wait()
    od[_NFIELD - 1].wait()


@jax.jit
def kernel(idx, value, table):
    idxT = idx.T
    valT = value.T
    mesh = plsc.VectorSubcoreMesh(core_axis_name="c", subcore_axis_name="s")
    tab_lin = pl.kernel(
        _detile_body,
        out_type=jax.ShapeDtypeStruct((_NFEAT * _NEMB,), jnp.float32),
        mesh=mesh,
        scratch_types=[
            pltpu.VMEM((2, _NEMB, _DBW), jnp.float32),
            pltpu.VMEM((_DBW * _NEMB,), jnp.float32),
            pltpu.VMEM((_DBW * _NEMB,), jnp.float32),
            pltpu.SemaphoreType.DMA((2,)),
            pltpu.SemaphoreType.DMA((2,)),
        ],
        compiler_params=pltpu.CompilerParams(
            use_tc_tiling_on_sc=True, needs_layout_passes=False
        ),
    )(table.T, lax.slice(table, (_TAIL_C0, 0), (_NFEAT, _NEMB)).reshape(-1))
    out = pl.kernel(
        _gather_body,
        out_type=jax.ShapeDtypeStruct((_NFIELD, _NEMB, _BATCH), jnp.float32),
        mesh=mesh,
        scratch_types=[
            pltpu.VMEM((_NFIELD, _BW), jnp.int32),
            pltpu.VMEM((_NFIELD, _BW), jnp.float32),
            pltpu.VMEM((2, _BW, _NEMB), jnp.float32),
            pltpu.VMEM((2, _NEMB, _BW), jnp.float32),
            pltpu.SemaphoreType.DMA((2,)),
            pltpu.SemaphoreType.DMA((2,)),
        ],
        compiler_params=pltpu.CompilerParams(
            use_tc_tiling_on_sc=False, needs_layout_passes=False
        ),
    )(idxT, valT, tab_lin.reshape(_NFEAT, _NEMB))
    return out.transpose(2, 0, 1)


# speedup vs baseline: 1.0765x; 1.0225x over previous
import functools

import jax
import jax.numpy as jnp
from jax import lax
from jax.experimental import pallas as pl
from jax.experimental.pallas import tpu as pltpu
from jax.experimental.pallas import tpu_sc as plsc

_NFIELD = 26
_NEMB = 16
_BATCH = 16384

_NW = 32
_BW = _BATCH // _NW
_NGRP = _BW // _NEMB

_NFEAT = 1000000
_DBW = 128
_NBLK_FULL = _NFEAT // _DBW
_BPW = _NBLK_FULL // _NW
_NEXTRA = _NBLK_FULL - _BPW * _NW
_TAIL_C0 = _NBLK_FULL * _DBW


def _detile_body(tabT_hbm, tail_hbm, out_hbm, buf_v, row0_v, row1_v, row2_v, row3_v, sem_i, sem_o):
    c = lax.axis_index("c")
    s = lax.axis_index("s")
    wid = s * 2 + c
    blk0 = wid * _BPW
    iota16 = lax.broadcasted_iota(jnp.int32, (16,), 0)
    rows = (row0_v, row1_v, row2_v, row3_v)

    def fire_in(i, b):
        c0 = (blk0 + i) * _DBW
        return pltpu.async_copy(
            tabT_hbm.at[:, pl.ds(c0, _DBW)], buf_v.at[b], sem_i.at[b]
        )

    def transpose_block(src, dst, width):
        @plsc.parallel_loop(0, width, step=1, unroll=4)
        def _(j):
            col = plsc.load_gather(src, [iota16, jnp.full((16,), j, jnp.int32)])
            dst[pl.ds(j * _NEMB, _NEMB)] = col

    for b in range(4):
        fire_in(b, b)

    def body2(t, _):
        for b in range(4):
            i = t * 4 + b
            c0 = (blk0 + i) * _DBW
            pltpu.make_async_copy(
                tabT_hbm.at[:, pl.ds(0, _DBW)], buf_v.at[b], sem_i.at[b]
            ).wait()

            @pl.when(i >= 4)
            def _():
                pltpu.make_async_copy(
                    rows[b], out_hbm.at[pl.ds(0, _DBW * _NEMB)], sem_o.at[b]
                ).wait()

            transpose_block(buf_v.at[b], rows[b], _DBW)

            pltpu.async_copy(
                rows[b], out_hbm.at[pl.ds(c0 * _NEMB, _DBW * _NEMB)], sem_o.at[b]
            )

            @pl.when(i + 4 < _BPW)
            def _():
                fire_in(i + 4, b)

        return 0

    assert _BPW % 4 == 0
    lax.fori_loop(0, _BPW // 4, body2, 0, unroll=False)
    if False:
        i_last = _BPW - 1
        b_last = i_last % 2
        pltpu.make_async_copy(
            tabT_hbm.at[:, pl.ds(0, _DBW)], buf_v.at[b_last], sem_i.at[b_last]
        ).wait()
        pltpu.make_async_copy(
            rows[b_last], out_hbm.at[pl.ds(0, _DBW * _NEMB)], sem_o.at[b_last]
        ).wait()
        transpose_block(buf_v.at[b_last], rows[b_last], _DBW)
        c0_last = (blk0 + i_last) * _DBW
        pltpu.async_copy(
            rows[b_last], out_hbm.at[pl.ds(c0_last * _NEMB, _DBW * _NEMB)],
            sem_o.at[b_last],
        )
        pltpu.make_async_copy(
            rows[1 - b_last], out_hbm.at[pl.ds(0, _DBW * _NEMB)],
            sem_o.at[1 - b_last],
        ).wait()
        pltpu.make_async_copy(
            rows[b_last], out_hbm.at[pl.ds(0, _DBW * _NEMB)], sem_o.at[b_last]
        ).wait()
    else:
        for b in range(4):
            pltpu.make_async_copy(
                rows[b], out_hbm.at[pl.ds(0, _DBW * _NEMB)], sem_o.at[b]
            ).wait()

    @pl.when(wid < _NEXTRA)
    def _():
        c0 = (_NBLK_FULL - _NEXTRA + wid) * _DBW
        pltpu.sync_copy(tabT_hbm.at[:, pl.ds(c0, _DBW)], buf_v.at[0])
        transpose_block(buf_v.at[0], row0_v, _DBW)
        pltpu.sync_copy(row0_v, out_hbm.at[pl.ds(c0 * _NEMB, _DBW * _NEMB)])

    @pl.when(wid == _NEXTRA)
    def _():
        tail_sz = (_NFEAT - _TAIL_C0) * _NEMB
        pltpu.sync_copy(tail_hbm, row0_v.at[pl.ds(0, tail_sz)])
        pltpu.sync_copy(
            row0_v.at[pl.ds(0, tail_sz)],
            out_hbm.at[pl.ds(_TAIL_C0 * _NEMB, tail_sz)],
        )


def _gather_body(idxT_hbm, valT_hbm, table_hbm, out_hbm,
                 idx_v, val_v, rows_v, col_v, sem_g, sem_o):
    c = lax.axis_index("c")
    s = lax.axis_index("s")
    wid = s * 2 + c
    b0 = wid * _BW

    pltpu.sync_copy(idxT_hbm.at[:, pl.ds(b0, _BW)], idx_v)
    pltpu.sync_copy(valT_hbm.at[:, pl.ds(b0, _BW)], val_v)

    iota16 = lax.broadcasted_iota(jnp.int32, (16,), 0)

    def fire_gather(f):
        b = f % 2
        return [
            pltpu.async_copy(
                table_hbm.at[idx_v.at[f, pl.ds(k * 128, 128)]],
                rows_v.at[b, pl.ds(k * 128, 128)],
                sem_g.at[b],
            )
            for k in range(_BW // 128)
        ]

    gd = {0: fire_gather(0)}
    od = {}
    for f in range(_NFIELD):
        b = f % 2
        if f + 1 < _NFIELD:
            gd[f + 1] = fire_gather(f + 1)
        for d in gd[f]:
            d.wait()
        if f >= 2:
            od[f - 2].wait()

        rows_b = rows_v.at[b]
        col_b = col_v.at[b]

        @plsc.parallel_loop(0, _NGRP, step=1, unroll=1)
        def _(j, f=f, rows_b=rows_b, col_b=col_b):
            r0 = j * _NEMB
            vals = val_v[f, pl.ds(r0, _NEMB)]
            ridx = iota16 + r0
            for e in range(_NEMB):
                col = plsc.load_gather(rows_b, [ridx, jnp.full((16,), e, jnp.int32)])
                col_b[e, pl.ds(r0, _NEMB)] = col * vals
        od[f] = pltpu.async_copy(
            col_b, out_hbm.at[f, :, pl.ds(b0, _BW)], sem_o.at[b]
        )
    od[_NFIELD - 2].wait()
    od[_NFIELD - 1].wait()


@jax.jit
def kernel(idx, value, table):
    idxT = idx.T
    valT = value.T
    mesh = plsc.VectorSubcoreMesh(core_axis_name="c", subcore_axis_name="s")
    tab_lin = pl.kernel(
        _detile_body,
        out_type=jax.ShapeDtypeStruct((_NFEAT * _NEMB,), jnp.float32),
        mesh=mesh,
        scratch_types=[
            pltpu.VMEM((4, _NEMB, _DBW), jnp.float32),
            pltpu.VMEM((_DBW * _NEMB,), jnp.float32),
            pltpu.VMEM((_DBW * _NEMB,), jnp.float32),
            pltpu.VMEM((_DBW * _NEMB,), jnp.float32),
            pltpu.VMEM((_DBW * _NEMB,), jnp.float32),
            pltpu.SemaphoreType.DMA((4,)),
            pltpu.SemaphoreType.DMA((4,)),
        ],
        compiler_params=pltpu.CompilerParams(
            use_tc_tiling_on_sc=True, needs_layout_passes=False
        ),
    )(table.T, lax.slice(table, (_TAIL_C0, 0), (_NFEAT, _NEMB)).reshape(-1))
    out = pl.kernel(
        _gather_body,
        out_type=jax.ShapeDtypeStruct((_NFIELD, _NEMB, _BATCH), jnp.float32),
        mesh=mesh,
        scratch_types=[
            pltpu.VMEM((_NFIELD, _BW), jnp.int32),
            pltpu.VMEM((_NFIELD, _BW), jnp.float32),
            pltpu.VMEM((2, _BW, _NEMB), jnp.float32),
            pltpu.VMEM((2, _NEMB, _BW), jnp.float32),
            pltpu.SemaphoreType.DMA((2,)),
            pltpu.SemaphoreType.DMA((2,)),
        ],
        compiler_params=pltpu.CompilerParams(
            use_tc_tiling_on_sc=False, needs_layout_passes=False
        ),
    )(idxT, valT, tab_lin.reshape(_NFEAT, _NEMB))
    return out.transpose(2, 0, 1)


# speedup vs baseline: 1.0772x; 1.0007x over previous
import jax
import jax.numpy as jnp
from jax import lax
from jax.experimental import pallas as pl
from jax.experimental.pallas import tpu as pltpu
from jax.experimental.pallas import tpu_sc as plsc

_NFIELD = 26
_NEMB = 16
_BATCH = 16384

_NW = 32
_BW = _BATCH // _NW
_NGRP = _BW // _NEMB

_NFEAT = 1000000
_DBW = 128
_NBLK_FULL = _NFEAT // _DBW
_BPW = _NBLK_FULL // _NW
_NEXTRA = _NBLK_FULL - _BPW * _NW
_TAIL_C0 = _NBLK_FULL * _DBW


def _detile_body(tabT_hbm, tail_hbm, out_hbm, buf_v, row0_v, row1_v, row2_v, row3_v, sem_i, sem_o):
    c = lax.axis_index("c")
    s = lax.axis_index("s")
    wid = s * 2 + c
    blk0 = wid * _BPW
    iota16 = lax.broadcasted_iota(jnp.int32, (16,), 0)
    rows = (row0_v, row1_v, row2_v, row3_v)

    def fire_in(i, b):
        c0 = (blk0 + i) * _DBW
        return pltpu.async_copy(
            tabT_hbm.at[:, pl.ds(c0, _DBW)], buf_v.at[b], sem_i.at[b]
        )

    def transpose_block(src, dst, width):
        @plsc.parallel_loop(0, width, step=1, unroll=4)
        def _(j):
            col = plsc.load_gather(src, [iota16, jnp.full((16,), j, jnp.int32)])
            dst[pl.ds(j * _NEMB, _NEMB)] = col

    for b in range(4):
        fire_in(b, b)

    def body2(t, _):
        for b in range(4):
            i = t * 4 + b
            c0 = (blk0 + i) * _DBW
            pltpu.make_async_copy(
                tabT_hbm.at[:, pl.ds(0, _DBW)], buf_v.at[b], sem_i.at[b]
            ).wait()

            @pl.when(i >= 4)
            def _():
                pltpu.make_async_copy(
                    rows[b], out_hbm.at[pl.ds(0, _DBW * _NEMB)], sem_o.at[b]
                ).wait()

            transpose_block(buf_v.at[b], rows[b], _DBW)

            pltpu.async_copy(
                rows[b], out_hbm.at[pl.ds(c0 * _NEMB, _DBW * _NEMB)], sem_o.at[b]
            )

            @pl.when(i + 4 < _BPW)
            def _():
                fire_in(i + 4, b)

        return 0

    assert _BPW % 4 == 0
    lax.fori_loop(0, _BPW // 4, body2, 0, unroll=False)
    for b in range(4):
        pltpu.make_async_copy(
            rows[b], out_hbm.at[pl.ds(0, _DBW * _NEMB)], sem_o.at[b]
        ).wait()

    @pl.when(wid < _NEXTRA)
    def _():
        c0 = (_NBLK_FULL - _NEXTRA + wid) * _DBW
        pltpu.sync_copy(tabT_hbm.at[:, pl.ds(c0, _DBW)], buf_v.at[0])
        transpose_block(buf_v.at[0], row0_v, _DBW)
        pltpu.sync_copy(row0_v, out_hbm.at[pl.ds(c0 * _NEMB, _DBW * _NEMB)])

    @pl.when(wid == _NEXTRA)
    def _():
        tail_sz = (_NFEAT - _TAIL_C0) * _NEMB
        pltpu.sync_copy(tail_hbm, row0_v.at[pl.ds(0, tail_sz)])
        pltpu.sync_copy(
            row0_v.at[pl.ds(0, tail_sz)],
            out_hbm.at[pl.ds(_TAIL_C0 * _NEMB, tail_sz)],
        )


def _gather_body(idxT_hbm, valT_hbm, table_hbm, out_hbm,
                 idx_v, val_v, rows_v, col_v, sem_g, sem_o):
    c = lax.axis_index("c")
    s = lax.axis_index("s")
    wid = s * 2 + c
    b0 = wid * _BW

    pltpu.sync_copy(idxT_hbm.at[:, pl.ds(b0, _BW)], idx_v)
    pltpu.sync_copy(valT_hbm.at[:, pl.ds(b0, _BW)], val_v)

    iota16 = lax.broadcasted_iota(jnp.int32, (16,), 0)

    def fire_gather(f):
        b = f % 2
        return [
            pltpu.async_copy(
                table_hbm.at[idx_v.at[f, pl.ds(k * 128, 128)]],
                rows_v.at[b, pl.ds(k * 128, 128)],
                sem_g.at[b],
            )
            for k in range(_BW // 128)
        ]

    gd = {0: fire_gather(0)}
    od = {}
    for f in range(_NFIELD):
        b = f % 2
        if f + 1 < _NFIELD:
            gd[f + 1] = fire_gather(f + 1)
        for d in gd[f]:
            d.wait()
        if f >= 2:
            od[f - 2].wait()

        rows_b = rows_v.at[b]
        col_b = col_v.at[b]

        @plsc.parallel_loop(0, _NGRP, step=1, unroll=1)
        def _(j, f=f, rows_b=rows_b, col_b=col_b):
            r0 = j * _NEMB
            vals = val_v[f, pl.ds(r0, _NEMB)]
            ridx = iota16 + r0
            for e in range(_NEMB):
                col = plsc.load_gather(rows_b, [ridx, jnp.full((16,), e, jnp.int32)])
                col_b[e, pl.ds(r0, _NEMB)] = col * vals
        od[f] = pltpu.async_copy(
            col_b, out_hbm.at[f, :, pl.ds(b0, _BW)], sem_o.at[b]
        )
    od[_NFIELD - 2].wait()
    od[_NFIELD - 1].wait()


@jax.jit
def kernel(idx, value, table):
    idxT = idx.T
    valT = value.T
    mesh = plsc.VectorSubcoreMesh(core_axis_name="c", subcore_axis_name="s")
    tab_lin = pl.kernel(
        _detile_body,
        out_type=jax.ShapeDtypeStruct((_NFEAT * _NEMB,), jnp.float32),
        mesh=mesh,
        scratch_types=[
            pltpu.VMEM((4, _NEMB, _DBW), jnp.float32),
            pltpu.VMEM((_DBW * _NEMB,), jnp.float32),
            pltpu.VMEM((_DBW * _NEMB,), jnp.float32),
            pltpu.VMEM((_DBW * _NEMB,), jnp.float32),
            pltpu.VMEM((_DBW * _NEMB,), jnp.float32),
            pltpu.SemaphoreType.DMA((4,)),
            pltpu.SemaphoreType.DMA((4,)),
        ],
        compiler_params=pltpu.CompilerParams(
            use_tc_tiling_on_sc=True, needs_layout_passes=False
        ),
    )(table.T, lax.slice(table, (_TAIL_C0, 0), (_NFEAT, _NEMB)).reshape(-1))
    out = pl.kernel(
        _gather_body,
        out_type=jax.ShapeDtypeStruct((_NFIELD, _NEMB, _BATCH), jnp.float32),
        mesh=mesh,
        scratch_types=[
            pltpu.VMEM((_NFIELD, _BW), jnp.int32),
            pltpu.VMEM((_NFIELD, _BW), jnp.float32),
            pltpu.VMEM((2, _BW, _NEMB), jnp.float32),
            pltpu.VMEM((2, _NEMB, _BW), jnp.float32),
            pltpu.SemaphoreType.DMA((2,)),
            pltpu.SemaphoreType.DMA((2,)),
        ],
        compiler_params=pltpu.CompilerParams(
            use_tc_tiling_on_sc=False, needs_layout_passes=False
        ),
    )(idxT, valT, tab_lin.reshape(_NFEAT, _NEMB))
    return out.transpose(2, 0, 1)
